# baseline (speedup 1.0000x reference)
import functools

import jax
import jax.numpy as jnp
from jax import lax
from jax.experimental import pallas as pl
from jax.experimental.pallas import tpu as pltpu
from jax.experimental.pallas import tpu_sc as plsc

IN_SIZE = 4096
OUT_SIZE = 4096
BATCH = 1024
NNZ = 1677721

SIZES = (410, 410, 410, 410, 408)
PREF = (0, 410, 820, 1230, 1640)
MAX_CW = max(SIZES) * IN_SIZE

WSZ = 2048
N_TILES = 16
WINDOWS = -(-NNZ // (N_TILES * WSZ))
PER_TILE = WINDOWS * WSZ
NNZ_PAD = PER_TILE * N_TILES
ZBUF = 4096

BN = 512


def _matmul_body(x_ref, w_ref, o_ref):
    o_ref[...] = lax.dot_general(
        x_ref[...].astype(jnp.bfloat16), w_ref[...].astype(jnp.bfloat16),
        dimension_numbers=(((1,), (1,)), ((), ())),
        preferred_element_type=jnp.float32,
    )


def _tc_matmul(x, w):
    return pl.pallas_call(
        _matmul_body,
        grid=(OUT_SIZE // BN,),
        in_specs=[
            pl.BlockSpec((BATCH, IN_SIZE), lambda j: (0, 0)),
            pl.BlockSpec((BN, IN_SIZE), lambda j: (j, 0)),
        ],
        out_specs=pl.BlockSpec((BATCH, BN), lambda j: (0, j)),
        out_shape=jax.ShapeDtypeStruct((BATCH, OUT_SIZE), jnp.float32),
    )(x, w)


PACK_WINDOWS = NNZ_PAD // (32 * WSZ)


@functools.partial(
    pl.kernel,
    out_type=jax.ShapeDtypeStruct((NNZ_PAD,), jnp.int32),
    mesh=plsc.VectorSubcoreMesh(core_axis_name="c", subcore_axis_name="s"),
    scratch_types=[
        pltpu.VMEM((2, WSZ), jnp.int32),
        pltpu.VMEM((2, WSZ), jnp.int32),
        pltpu.VMEM((WSZ,), jnp.int32),
        pltpu.VMEM((WSZ,), jnp.int32),
        pltpu.SemaphoreType.DMA((2,)),
        pltpu.SemaphoreType.DMA((2,)),
    ],
)
def _sc_pack(rows_hbm, cols_hbm, g_hbm, rows_v, cols_v, g0, g1, st_sem, out_sem):
    c = lax.axis_index("c")
    s = lax.axis_index("s")
    wid = s * 2 + c
    g_b = (g0, g1)

    def stage_start(w, b):
        base = (wid * PACK_WINDOWS + w) * WSZ
        pltpu.async_copy(rows_hbm.at[pl.ds(base, WSZ)], rows_v.at[b], st_sem.at[b])
        pltpu.async_copy(cols_hbm.at[pl.ds(base, WSZ)], cols_v.at[b], st_sem.at[b])

    def stage_wait(w, b):
        base = (wid * PACK_WINDOWS + w) * WSZ
        pltpu.make_async_copy(rows_hbm.at[pl.ds(base, WSZ)], rows_v.at[b], st_sem.at[b]).wait()
        pltpu.make_async_copy(cols_hbm.at[pl.ds(base, WSZ)], cols_v.at[b], st_sem.at[b]).wait()

    def do_window(w, b):
        stage_wait(w, b)

        def vec(i, _):
            sl = pl.ds(i * 16, 16)
            g_b[b][sl] = (rows_v[b, sl] << 12) | cols_v[b, sl]
            return 0
        lax.fori_loop(0, WSZ // 16, vec, 0)
        base = (wid * PACK_WINDOWS + w) * WSZ
        pltpu.async_copy(g_b[b], g_hbm.at[pl.ds(base, WSZ)], out_sem.at[b])

    def out_wait(w, b):
        base = (wid * PACK_WINDOWS + w) * WSZ
        pltpu.make_async_copy(g_b[b], g_hbm.at[pl.ds(base, WSZ)], out_sem.at[b]).wait()

    for b in range(2):
        stage_start(b, b)
    do_window(0, 0)
    stage_start(2, 0)
    do_window(1, 1)
    stage_start(3, 1)

    def w2(i, _):
        for b in range(2):
            w = 2 + 2 * i + b
            out_wait(w - 2, b)
            do_window(w, b)
            nxt = jnp.minimum(w + 2, PACK_WINDOWS - 1)
            pl.when(w + 2 < PACK_WINDOWS)(lambda: stage_start(nxt, b))
        return 0
    lax.fori_loop(0, (PACK_WINDOWS - 2) // 2, w2, 0)
    for b in range(2):
        out_wait(PACK_WINDOWS - 2 + b, b)


@functools.partial(
    pl.kernel,
    out_type=jax.ShapeDtypeStruct((OUT_SIZE * IN_SIZE,), jnp.float32),
    mesh=plsc.VectorSubcoreMesh(core_axis_name="c", subcore_axis_name="s"),
    scratch_types=[
        pltpu.VMEM((2, WSZ), jnp.int32),
        pltpu.VMEM((2, WSZ), jnp.int32),
        pltpu.VMEM((2, WSZ), jnp.float32),
        pltpu.VMEM((WSZ,), jnp.int32),
        pltpu.VMEM((WSZ,), jnp.int32),
        pltpu.VMEM((WSZ,), jnp.float32),
        pltpu.VMEM((WSZ,), jnp.float32),
        pltpu.VMEM((ZBUF,), jnp.float32),
        pltpu.VMEM_SHARED((MAX_CW,), jnp.float32),
        pltpu.SemaphoreType.DMA((2,)),
        pltpu.SemaphoreType.DMA((2,)),
    ],
)
def _sc_build_w(rows_hbm, cols_hbm, vals_hbm, w_hbm,
                rows_v, cols_v, vals_v, idx_v0, idx_v1, val_v0, val_v1,
                zero_v, chunk_sp, st_sem, sc_sem):
    c = lax.axis_index("c")
    s = lax.axis_index("s")
    idx_b = (idx_v0, idx_v1)
    val_b = (val_v0, val_v1)

    def zinit(i, _):
        zero_v[pl.ds(i * 16, 16)] = jnp.zeros((16,), jnp.float32)
        return 0
    lax.fori_loop(0, ZBUF // 16, zinit, 0)

    def stage_start(w, b):
        base = s * PER_TILE + w * WSZ
        pltpu.async_copy(rows_hbm.at[pl.ds(base, WSZ)], rows_v.at[b], st_sem.at[b])
        pltpu.async_copy(cols_hbm.at[pl.ds(base, WSZ)], cols_v.at[b], st_sem.at[b])
        pltpu.async_copy(vals_hbm.at[pl.ds(base, WSZ)], vals_v.at[b], st_sem.at[b])

    def stage_wait(w, b):
        base = s * PER_TILE + w * WSZ
        pltpu.make_async_copy(rows_hbm.at[pl.ds(base, WSZ)], rows_v.at[b], st_sem.at[b]).wait()
        pltpu.make_async_copy(cols_hbm.at[pl.ds(base, WSZ)], cols_v.at[b], st_sem.at[b]).wait()
        pltpu.make_async_copy(vals_hbm.at[pl.ds(base, WSZ)], vals_v.at[b], st_sem.at[b]).wait()

    def scatter_start(b):
        pltpu.async_copy(val_b[b], chunk_sp.at[idx_b[b]], sc_sem.at[b], add=True)

    def scatter_wait(b):
        pltpu.make_async_copy(val_b[b], chunk_sp.at[idx_b[b]], sc_sem.at[b]).wait()

    for p in range(len(SIZES)):
        cw = SIZES[p] * IN_SIZE
        share = cw // N_TILES
        lo = c * (2048 * IN_SIZE) + PREF[p] * IN_SIZE

        def compute(b, lo=lo, cw=cw):
            def vec(i, _):
                sl = pl.ds(i * 16, 16)
                g = (rows_v[b, sl] << 12) | cols_v[b, sl]
                rel = g - lo
                m = (rel >= 0) & (rel < cw)
                idx_b[b][sl] = jnp.where(m, rel, g & 0x3FFFF)
                val_b[b][sl] = jnp.where(m, vals_v[b, sl], 0.0)
                return 0
            lax.fori_loop(0, WSZ // 16, vec, 0)

        for k in range(share // ZBUF):
            pltpu.sync_copy(zero_v, chunk_sp.at[pl.ds(s * share + k * ZBUF, ZBUF)])
        rem = share % ZBUF
        if rem:
            pltpu.sync_copy(zero_v.at[pl.ds(0, rem)],
                            chunk_sp.at[pl.ds(s * share + (share // ZBUF) * ZBUF, rem)])
        plsc.subcore_barrier()

        for b in range(2):
            stage_start(b, b)
        for b in range(2):
            stage_wait(b, b)
            compute(b)
            scatter_start(b)
            stage_start(2 + b, b)

        def window2(w2, _):
            for b in range(2):
                w = w2 + b
                stage_wait(w, b)
                scatter_wait(b)
                compute(b)
                scatter_start(b)
                nxt = jnp.minimum(w + 2, WINDOWS - 1)
                pl.when(w + 2 < WINDOWS)(lambda: stage_start(nxt, b))
            return 0
        lax.fori_loop(0, (WINDOWS - 2) // 2, lambda i, u: window2(2 + 2 * i, u), 0)
        for b in range(2):
            scatter_wait(b)
        plsc.subcore_barrier()

        pltpu.sync_copy(
            chunk_sp.at[pl.ds(s * share, share)],
            w_hbm.at[pl.ds(lo + s * share, share)],
        )
        plsc.subcore_barrier()


def kernel(x, rows, cols, vals):
    pad = NNZ_PAD - NNZ
    rows_p = jnp.concatenate([rows.astype(jnp.int32), jnp.zeros((pad,), jnp.int32)])
    cols_p = jnp.concatenate([cols.astype(jnp.int32), jnp.zeros((pad,), jnp.int32)])
    vals_p = jnp.concatenate([vals, jnp.zeros((pad,), jnp.float32)])
    w_flat = _sc_build_w(rows_p, cols_p, vals_p)
    W = w_flat.reshape(OUT_SIZE, IN_SIZE)
    return _tc_matmul(x, W)


# speedup vs baseline: 1.0118x; 1.0118x over previous
import functools

import jax
import jax.numpy as jnp
from jax import lax
from jax.experimental import pallas as pl
from jax.experimental.pallas import tpu as pltpu
from jax.experimental.pallas import tpu_sc as plsc

IN_SIZE = 4096
OUT_SIZE = 4096
BATCH = 1024
NNZ = 1677721

SIZES = (410, 410, 410, 410, 408)
PREF = (0, 410, 820, 1230, 1640)
MAX_CW = max(SIZES) * IN_SIZE

WSZ = 2048
N_TILES = 16
WINDOWS = -(-NNZ // (N_TILES * WSZ))
PER_TILE = WINDOWS * WSZ
NNZ_PAD = PER_TILE * N_TILES
ZBUF = 8192

BN = 512


def _matmul_body(x_ref, w_ref, o_ref):
    o_ref[...] = lax.dot_general(
        x_ref[...].astype(jnp.bfloat16), w_ref[...].astype(jnp.bfloat16),
        dimension_numbers=(((1,), (1,)), ((), ())),
        preferred_element_type=jnp.float32,
    )


def _tc_matmul(x, w):
    return pl.pallas_call(
        _matmul_body,
        grid=(OUT_SIZE // BN,),
        in_specs=[
            pl.BlockSpec((BATCH, IN_SIZE), lambda j: (0, 0)),
            pl.BlockSpec((BN, IN_SIZE), lambda j: (j, 0)),
        ],
        out_specs=pl.BlockSpec((BATCH, BN), lambda j: (0, j)),
        out_shape=jax.ShapeDtypeStruct((BATCH, OUT_SIZE), jnp.float32),
    )(x, w)


PACK_WINDOWS = NNZ_PAD // (32 * WSZ)


@functools.partial(
    pl.kernel,
    out_type=jax.ShapeDtypeStruct((NNZ_PAD,), jnp.int32),
    mesh=plsc.VectorSubcoreMesh(core_axis_name="c", subcore_axis_name="s"),
    scratch_types=[
        pltpu.VMEM((2, WSZ), jnp.int32),
        pltpu.VMEM((2, WSZ), jnp.int32),
        pltpu.VMEM((WSZ,), jnp.int32),
        pltpu.VMEM((WSZ,), jnp.int32),
        pltpu.SemaphoreType.DMA((2,)),
        pltpu.SemaphoreType.DMA((2,)),
    ],
)
def _sc_pack(rows_hbm, cols_hbm, g_hbm, rows_v, cols_v, g0, g1, st_sem, out_sem):
    c = lax.axis_index("c")
    s = lax.axis_index("s")
    wid = s * 2 + c
    g_b = (g0, g1)

    def stage_start(w, b):
        base = (wid * PACK_WINDOWS + w) * WSZ
        pltpu.async_copy(rows_hbm.at[pl.ds(base, WSZ)], rows_v.at[b], st_sem.at[b])
        pltpu.async_copy(cols_hbm.at[pl.ds(base, WSZ)], cols_v.at[b], st_sem.at[b])

    def stage_wait(w, b):
        base = (wid * PACK_WINDOWS + w) * WSZ
        pltpu.make_async_copy(rows_hbm.at[pl.ds(base, WSZ)], rows_v.at[b], st_sem.at[b]).wait()
        pltpu.make_async_copy(cols_hbm.at[pl.ds(base, WSZ)], cols_v.at[b], st_sem.at[b]).wait()

    def do_window(w, b):
        stage_wait(w, b)

        def vec(i, _):
            sl = pl.ds(i * 16, 16)
            g_b[b][sl] = (rows_v[b, sl] << 12) | cols_v[b, sl]
            return 0
        lax.fori_loop(0, WSZ // 16, vec, 0)
        base = (wid * PACK_WINDOWS + w) * WSZ
        pltpu.async_copy(g_b[b], g_hbm.at[pl.ds(base, WSZ)], out_sem.at[b])

    def out_wait(w, b):
        base = (wid * PACK_WINDOWS + w) * WSZ
        pltpu.make_async_copy(g_b[b], g_hbm.at[pl.ds(base, WSZ)], out_sem.at[b]).wait()

    for b in range(2):
        stage_start(b, b)
    do_window(0, 0)
    stage_start(2, 0)
    do_window(1, 1)
    stage_start(3, 1)

    def w2(i, _):
        for b in range(2):
            w = 2 + 2 * i + b
            out_wait(w - 2, b)
            do_window(w, b)
            nxt = jnp.minimum(w + 2, PACK_WINDOWS - 1)
            pl.when(w + 2 < PACK_WINDOWS)(lambda: stage_start(nxt, b))
        return 0
    lax.fori_loop(0, (PACK_WINDOWS - 2) // 2, w2, 0)
    for b in range(2):
        out_wait(PACK_WINDOWS - 2 + b, b)


@functools.partial(
    pl.kernel,
    out_type=jax.ShapeDtypeStruct((OUT_SIZE * IN_SIZE,), jnp.float32),
    mesh=plsc.VectorSubcoreMesh(core_axis_name="c", subcore_axis_name="s"),
    scratch_types=[
        pltpu.VMEM((2, WSZ), jnp.int32),
        pltpu.VMEM((2, WSZ), jnp.float32),
        pltpu.VMEM((WSZ,), jnp.int32),
        pltpu.VMEM((WSZ,), jnp.int32),
        pltpu.VMEM((WSZ,), jnp.float32),
        pltpu.VMEM((WSZ,), jnp.float32),
        pltpu.VMEM((ZBUF,), jnp.float32),
        pltpu.VMEM_SHARED((MAX_CW,), jnp.float32),
        pltpu.SemaphoreType.DMA((2,)),
        pltpu.SemaphoreType.DMA((2,)),
    ],
)
def _sc_build_w(g_hbm, vals_hbm, w_hbm,
                g_v, vals_v, idx_v0, idx_v1, val_v0, val_v1,
                zero_v, chunk_sp, st_sem, sc_sem):
    c = lax.axis_index("c")
    s = lax.axis_index("s")
    idx_b = (idx_v0, idx_v1)
    val_b = (val_v0, val_v1)

    def zinit(i, _):
        zero_v[pl.ds(i * 16, 16)] = jnp.zeros((16,), jnp.float32)
        return 0
    lax.fori_loop(0, ZBUF // 16, zinit, 0)

    def stage_start(w, b):
        base = s * PER_TILE + w * WSZ
        pltpu.async_copy(g_hbm.at[pl.ds(base, WSZ)], g_v.at[b], st_sem.at[b])
        pltpu.async_copy(vals_hbm.at[pl.ds(base, WSZ)], vals_v.at[b], st_sem.at[b])

    def stage_wait(w, b):
        base = s * PER_TILE + w * WSZ
        pltpu.make_async_copy(g_hbm.at[pl.ds(base, WSZ)], g_v.at[b], st_sem.at[b]).wait()
        pltpu.make_async_copy(vals_hbm.at[pl.ds(base, WSZ)], vals_v.at[b], st_sem.at[b]).wait()

    def scatter_start(b):
        pltpu.async_copy(val_b[b], chunk_sp.at[idx_b[b]], sc_sem.at[b], add=True)

    def scatter_wait(b):
        pltpu.make_async_copy(val_b[b], chunk_sp.at[idx_b[b]], sc_sem.at[b]).wait()

    for p in range(len(SIZES)):
        cw = SIZES[p] * IN_SIZE
        share = cw // N_TILES
        lo = c * (2048 * IN_SIZE) + PREF[p] * IN_SIZE

        def compute(b, lo=lo, cw=cw):
            def vec(i, _):
                sl = pl.ds(i * 16, 16)
                g = g_v[b, sl]
                rel = g - lo
                m = (rel >= 0) & (rel < cw)
                idx_b[b][sl] = jnp.where(m, rel, g & 0x3FFFF)
                val_b[b][sl] = jnp.where(m, vals_v[b, sl], 0.0)
                return 0
            lax.fori_loop(0, WSZ // 16, vec, 0)

        for k in range(share // ZBUF):
            pltpu.sync_copy(zero_v, chunk_sp.at[pl.ds(s * share + k * ZBUF, ZBUF)])
        rem = share % ZBUF
        if rem:
            pltpu.sync_copy(zero_v.at[pl.ds(0, rem)],
                            chunk_sp.at[pl.ds(s * share + (share // ZBUF) * ZBUF, rem)])
        plsc.subcore_barrier()

        for b in range(2):
            stage_start(b, b)
        for b in range(2):
            stage_wait(b, b)
            compute(b)
            scatter_start(b)
            stage_start(2 + b, b)

        def window2(w2, _):
            for b in range(2):
                w = w2 + b
                stage_wait(w, b)
                scatter_wait(b)
                compute(b)
                scatter_start(b)
                nxt = jnp.minimum(w + 2, WINDOWS - 1)
                pl.when(w + 2 < WINDOWS)(lambda: stage_start(nxt, b))
            return 0
        lax.fori_loop(0, (WINDOWS - 2) // 2, lambda i, u: window2(2 + 2 * i, u), 0)
        for b in range(2):
            scatter_wait(b)
        plsc.subcore_barrier()

        pltpu.sync_copy(
            chunk_sp.at[pl.ds(s * share, share)],
            w_hbm.at[pl.ds(lo + s * share, share)],
        )
        plsc.subcore_barrier()


def kernel(x, rows, cols, vals):
    pad = NNZ_PAD - NNZ
    rows_p = jnp.concatenate([rows.astype(jnp.int32), jnp.zeros((pad,), jnp.int32)])
    cols_p = jnp.concatenate([cols.astype(jnp.int32), jnp.zeros((pad,), jnp.int32)])
    vals_p = jnp.concatenate([vals, jnp.zeros((pad,), jnp.float32)])
    g = _sc_pack(rows_p, cols_p)
    w_flat = _sc_build_w(g, vals_p)
    W = w_flat.reshape(OUT_SIZE, IN_SIZE)
    return _tc_matmul(x, W)
